# baseline (device time: 8173 ns/iter reference)
import jax
import jax.numpy as jnp
from jax import lax
from jax.experimental import pallas as pl
from jax.experimental.pallas import tpu as pltpu

N_DEV = 4
N_ROUNDS = 2


def kernel(x):
    m_per, n = x.shape
    rows = m_per // N_ROUNDS

    def body(
        x_hbm, out_ref, xbuf, sbuf, comm_ref, copy_sems, send_sems, recv_sems
    ):
        my_pos = lax.axis_index("i")

        barrier_sem = pltpu.get_barrier_semaphore()
        for d in range(1, N_DEV):
            pl.semaphore_signal(
                barrier_sem,
                inc=1,
                device_id=((my_pos + d) % N_DEV,),
                device_id_type=pl.DeviceIdType.MESH,
            )

        copies = []
        for r in range(N_ROUNDS):
            cp = pltpu.make_async_copy(
                x_hbm.at[pl.ds(r * rows, rows), :],
                xbuf.at[pl.ds(r * rows, rows), :],
                copy_sems.at[r],
            )
            cp.start()
            copies.append(cp)

        sends = []
        for r in range(N_ROUNDS):
            copies[r].wait()
            sbuf[r] = jnp.max(
                xbuf[pl.ds(r * rows, rows), :], axis=0, keepdims=True
            )
            if r == 0:
                pl.semaphore_wait(barrier_sem, N_DEV - 1)
            for d in range(1, N_DEV):
                target = (my_pos + d) % N_DEV
                slot = N_DEV - 1 - d
                rdma = pltpu.make_async_remote_copy(
                    src_ref=sbuf.at[r],
                    dst_ref=comm_ref.at[r, slot],
                    send_sem=send_sems.at[r, slot],
                    recv_sem=recv_sems.at[r, slot],
                    device_id=(target,),
                    device_id_type=pl.DeviceIdType.MESH,
                )
                rdma.start()
                sends.append(rdma)

        for r in range(N_ROUNDS):
            for k in range(N_DEV - 1):
                recv = pltpu.make_async_remote_copy(
                    src_ref=sbuf.at[r],
                    dst_ref=comm_ref.at[r, k],
                    send_sem=send_sems.at[r, k],
                    recv_sem=recv_sems.at[r, k],
                    device_id=(my_pos,),
                    device_id_type=pl.DeviceIdType.MESH,
                )
                recv.wait_recv()

        for rdma in sends:
            rdma.wait_send()

        out_ref[...] = jnp.maximum(
            jnp.max(sbuf[...], axis=0),
            jnp.max(comm_ref[...], axis=(0, 1)),
        )

    return pl.pallas_call(
        body,
        out_shape=jax.ShapeDtypeStruct((1, n), jnp.float32),
        in_specs=[pl.BlockSpec(memory_space=pl.ANY)],
        out_specs=pl.BlockSpec(memory_space=pltpu.VMEM),
        scratch_shapes=[
            pltpu.VMEM((m_per, n), jnp.float32),
            pltpu.VMEM((N_ROUNDS, 1, n), jnp.float32),
            pltpu.VMEM((N_ROUNDS, N_DEV - 1, 1, n), jnp.float32),
            pltpu.SemaphoreType.DMA((N_ROUNDS,)),
            pltpu.SemaphoreType.DMA((N_ROUNDS, N_DEV - 1)),
            pltpu.SemaphoreType.DMA((N_ROUNDS, N_DEV - 1)),
        ],
        compiler_params=pltpu.CompilerParams(collective_id=0),
    )(x)


# device time: 7884 ns/iter; 1.0367x vs baseline; 1.0367x over previous
import jax
import jax.numpy as jnp
from jax import lax
from jax.experimental import pallas as pl
from jax.experimental.pallas import tpu as pltpu

N_DEV = 4


def kernel(x):
    m_per, n = x.shape

    def body(x_ref, out_ref, comm_ref, send_sems, recv_sems):
        my_pos = lax.axis_index("i")

        barrier_sem = pltpu.get_barrier_semaphore()
        for d in range(1, N_DEV):
            pl.semaphore_signal(
                barrier_sem,
                inc=1,
                device_id=((my_pos + d) % N_DEV,),
                device_id_type=pl.DeviceIdType.MESH,
            )

        out_ref[...] = jnp.max(x_ref[...], axis=0, keepdims=True)

        pl.semaphore_wait(barrier_sem, N_DEV - 1)

        sends = []
        for d in (2, 1, 3):
            target = (my_pos + d) % N_DEV
            slot = N_DEV - 1 - d
            rdma = pltpu.make_async_remote_copy(
                src_ref=out_ref,
                dst_ref=comm_ref.at[slot],
                send_sem=send_sems.at[slot],
                recv_sem=recv_sems.at[slot],
                device_id=(target,),
                device_id_type=pl.DeviceIdType.MESH,
            )
            rdma.start()
            sends.append(rdma)

        for k in (0, 2, 1):
            recv = pltpu.make_async_remote_copy(
                src_ref=out_ref,
                dst_ref=comm_ref.at[k],
                send_sem=send_sems.at[k],
                recv_sem=recv_sems.at[k],
                device_id=(my_pos,),
                device_id_type=pl.DeviceIdType.MESH,
            )
            recv.wait_recv()

        for rdma in sends:
            rdma.wait_send()

        out_ref[...] = jnp.maximum(
            out_ref[...], jnp.max(comm_ref[...], axis=0)
        )

    return pl.pallas_call(
        body,
        out_shape=jax.ShapeDtypeStruct((1, n), jnp.float32),
        in_specs=[pl.BlockSpec(memory_space=pltpu.VMEM)],
        out_specs=pl.BlockSpec(memory_space=pltpu.VMEM),
        scratch_shapes=[
            pltpu.VMEM((N_DEV - 1, 1, n), jnp.float32),
            pltpu.SemaphoreType.DMA((N_DEV - 1,)),
            pltpu.SemaphoreType.DMA((N_DEV - 1,)),
        ],
        compiler_params=pltpu.CompilerParams(collective_id=0),
    )(x)
